# baseline (speedup 1.0000x reference)
import jax
import jax.numpy as jnp
from jax import lax
from jax.experimental import pallas as pl
from jax.experimental.pallas import tpu as pltpu
from jax.experimental.pallas import tpu_sc as plsc

B = 16384
EMB = 32
K = 4
KD = EMB * K

NC = 2
NS = 16
NW = NC * NS
BPW = B // NW


IC = 128
NCHUNK = BPW // IC


def _sc_gather_body(uid_hbm, iid_hbm, uemb_hbm, iemb_hbm, ubias_hbm, ibias_hbm,
                    ue_out, ie_out, ub_out, ib_out,
                    uidx_v, iidx_v, ue_v, ie_v, ub_v, ib_v, sem):
    wid = lax.axis_index("s") * NC + lax.axis_index("c")
    base = wid * BPW
    pltpu.sync_copy(uid_hbm.at[pl.ds(wid * NCHUNK, NCHUNK)], uidx_v)
    pltpu.sync_copy(iid_hbm.at[pl.ds(wid * NCHUNK, NCHUNK)], iidx_v)
    copies = []
    for j in range(NCHUNK):
        r = pl.ds(j * IC, IC)
        copies.append(pltpu.make_async_copy(uemb_hbm.at[uidx_v.at[j]], ue_v.at[r], sem))
        copies.append(pltpu.make_async_copy(iemb_hbm.at[iidx_v.at[j]], ie_v.at[r], sem))
        copies.append(pltpu.make_async_copy(ubias_hbm.at[uidx_v.at[j]], ub_v.at[r], sem))
        copies.append(pltpu.make_async_copy(ibias_hbm.at[iidx_v.at[j]], ib_v.at[r], sem))
    for c in copies:
        c.start()
    for c in copies:
        c.wait()
    pltpu.sync_copy(ue_v, ue_out.at[pl.ds(base, BPW)])
    pltpu.sync_copy(ie_v, ie_out.at[pl.ds(base, BPW)])
    pltpu.sync_copy(ub_v, ub_out.at[pl.ds(base, BPW)])
    pltpu.sync_copy(ib_v, ib_out.at[pl.ds(base, BPW)])


def _sc_gather(uids, iids, user_emb, item_emb, user_bias, item_bias):
    mesh = plsc.VectorSubcoreMesh(core_axis_name="c", subcore_axis_name="s",
                                  num_cores=NC, num_subcores=NS)
    f = pl.kernel(
        _sc_gather_body,
        out_type=(
            jax.ShapeDtypeStruct((B, EMB), jnp.float32),
            jax.ShapeDtypeStruct((B, EMB), jnp.float32),
            jax.ShapeDtypeStruct((B, 1), jnp.float32),
            jax.ShapeDtypeStruct((B, 1), jnp.float32),
        ),
        mesh=mesh,
        compiler_params=pltpu.CompilerParams(use_tc_tiling_on_sc=False),
        scratch_types=[
            pltpu.VMEM((NCHUNK, IC), jnp.int32),
            pltpu.VMEM((NCHUNK, IC), jnp.int32),
            pltpu.VMEM((BPW, EMB), jnp.float32),
            pltpu.VMEM((BPW, EMB), jnp.float32),
            pltpu.VMEM((BPW, 1), jnp.float32),
            pltpu.VMEM((BPW, 1), jnp.float32),
            pltpu.SemaphoreType.DMA,
        ],
    )
    return f(uids.reshape(B // IC, IC), iids.reshape(B // IC, IC),
             user_emb, item_emb, user_bias, item_bias)


BLK = 2048


def _tc_mix_body(ue_ref, ie_ref, ub_ref, ib_ref, wt_ref, bt_ref, wa_ref, ba_ref,
                 out_ref):
    ue = ue_ref[...]
    ie = ie_ref[...]
    t = jnp.dot(ue, wt_ref[...], preferred_element_type=jnp.float32) + bt_ref[...]
    a = jnp.dot(ue, wa_ref[...], preferred_element_type=jnp.float32) + ba_ref[...]
    ie4 = jnp.concatenate([ie, ie, ie, ie], axis=1)
    q = t * ie4
    l = a * ie4
    ri = lax.broadcasted_iota(jnp.int32, (KD, KD), 0) // EMB
    ci = lax.broadcasted_iota(jnp.int32, (KD, KD), 1) // EMB
    s = (ri == ci).astype(jnp.float32)
    z = jnp.dot(l, s, preferred_element_type=jnp.float32)
    e = jnp.exp(z)
    denom = jnp.sum(e, axis=1, keepdims=True)
    num = jnp.sum(e * q, axis=1, keepdims=True)
    out_ref[...] = num * float(EMB) / denom + ub_ref[...] + ib_ref[...]


def _tc_mix(ue, ie, ub, ib, Wt, bt, Wa, ba):
    grid = (B // BLK,)
    return pl.pallas_call(
        _tc_mix_body,
        grid=grid,
        in_specs=[
            pl.BlockSpec((BLK, EMB), lambda i: (i, 0)),
            pl.BlockSpec((BLK, EMB), lambda i: (i, 0)),
            pl.BlockSpec((BLK, 1), lambda i: (i, 0)),
            pl.BlockSpec((BLK, 1), lambda i: (i, 0)),
            pl.BlockSpec((EMB, KD), lambda i: (0, 0)),
            pl.BlockSpec((1, KD), lambda i: (0, 0)),
            pl.BlockSpec((EMB, KD), lambda i: (0, 0)),
            pl.BlockSpec((1, KD), lambda i: (0, 0)),
        ],
        out_specs=pl.BlockSpec((BLK, 1), lambda i: (i, 0)),
        out_shape=jax.ShapeDtypeStruct((B, 1), jnp.float32),
    )(ue, ie, ub, ib, Wt, bt, Wa, ba)


@jax.jit
def kernel(user_ids, item_ids, user_emb, item_emb, user_bias, item_bias,
           Wt, bt, Wa, ba):
    uids = user_ids.astype(jnp.int32)
    iids = item_ids.astype(jnp.int32)
    ue, ie, ub, ib = _sc_gather(uids, iids, user_emb, item_emb,
                                user_bias, item_bias)
    out = _tc_mix(ue, ie, ub, ib, Wt, bt.reshape(1, KD), Wa, ba.reshape(1, KD))
    return out.reshape(-1)


# speedup vs baseline: 2.6209x; 2.6209x over previous
import jax
import jax.numpy as jnp
from jax import lax
from jax.experimental import pallas as pl
from jax.experimental.pallas import tpu as pltpu
from jax.experimental.pallas import tpu_sc as plsc

B = 16384
EMB = 32
K = 4
KD = EMB * K
NROWS = 1000000
PACK = KD // EMB
BROWS = NROWS // KD + 1

NC = 2
NS = 16
NW = NC * NS
BPW = B // NW
IC = 128
NCHUNK = BPW // IC


def _sc_gather_body(uidq_hbm, iidq_hbm, uemb_hbm, iemb_hbm, ubias_hbm, ibias_hbm,
                    ue_out, ie_out, ub_out, ib_out,
                    uidq_v, iidq_v, bidx_v, rows_v, sem):
    wid = lax.axis_index("s") * NC + lax.axis_index("c")
    base = wid * BPW
    pltpu.sync_copy(uidq_hbm, uidq_v)
    pltpu.sync_copy(iidq_hbm, iidq_v)
    for tbl, idx_v, out in ((uemb_hbm, uidq_v, ue_out), (iemb_hbm, iidq_v, ie_out)):
        copies = []
        for j in range(NCHUNK):
            copies.append(pltpu.make_async_copy(
                tbl.at[idx_v.at[wid * NCHUNK + j]],
                rows_v.at[pl.ds(j * IC, IC)], sem))
        for c in copies:
            c.start()
        for c in copies:
            c.wait()
        pltpu.sync_copy(rows_v, out.at[pl.ds(base, BPW)])
    for idx_v, k, tbl, out in ((uidq_v, 0, ubias_hbm, ub_out),
                               (iidq_v, 1, ibias_hbm, ib_out)):
        for j in range(NCHUNK):
            for t in range(IC // 16):
                q = idx_v[wid * NCHUNK + j, pl.ds(t * 16, 16)]
                bidx_v[j, pl.ds(t * 16, 16)] = jnp.right_shift(q, 5)
        copies = []
        for j in range(NCHUNK):
            copies.append(pltpu.make_async_copy(
                tbl.at[bidx_v.at[j]],
                rows_v.at[pl.ds(j * IC, IC)], sem))
        for c in copies:
            c.start()
        for c in copies:
            c.wait()
        pltpu.sync_copy(rows_v, out.at[pl.ds(base, BPW)])


def _sc_gather(uidq, iidq, uemb128, iemb128, ubias128, ibias128):
    mesh = plsc.VectorSubcoreMesh(core_axis_name="c", subcore_axis_name="s",
                                  num_cores=NC, num_subcores=NS)
    f = pl.kernel(
        _sc_gather_body,
        out_type=(
            jax.ShapeDtypeStruct((B, KD), jnp.float32),
            jax.ShapeDtypeStruct((B, KD), jnp.float32),
            jax.ShapeDtypeStruct((B, KD), jnp.float32),
            jax.ShapeDtypeStruct((B, KD), jnp.float32),
        ),
        mesh=mesh,
        compiler_params=pltpu.CompilerParams(use_tc_tiling_on_sc=True),
        scratch_types=[
            pltpu.VMEM((B // IC, IC), jnp.int32),
            pltpu.VMEM((B // IC, IC), jnp.int32),
            pltpu.VMEM((NCHUNK, IC), jnp.int32),
            pltpu.VMEM((BPW, KD), jnp.float32),
            pltpu.SemaphoreType.DMA,
        ],
    )
    return f(uidq, iidq, uemb128, iemb128, ubias128, ibias128)


BLK = 2048


def _tc_mix_body(ue4_ref, ie4_ref, uo_ref, io_ref, ul_ref, il_ref,
                 ub4_ref, ib4_ref, wt_ref, bt_ref, wa_ref, ba_ref, out_ref):
    uo = uo_ref[...]
    io = io_ref[...]
    ue4 = ue4_ref[...]
    ie4 = ie4_ref[...]
    ue = jnp.zeros((BLK, EMB), jnp.float32)
    ie = jnp.zeros((BLK, EMB), jnp.float32)
    for o in range(PACK):
        ue = ue + jnp.where(uo == o, ue4[:, o * EMB:(o + 1) * EMB], 0.0)
        ie = ie + jnp.where(io == o, ie4[:, o * EMB:(o + 1) * EMB], 0.0)
    lane = lax.broadcasted_iota(jnp.int32, (BLK, KD), 1)
    ub = jnp.sum(jnp.where(lane == ul_ref[...], ub4_ref[...], 0.0),
                 axis=1, keepdims=True)
    ib = jnp.sum(jnp.where(lane == il_ref[...], ib4_ref[...], 0.0),
                 axis=1, keepdims=True)
    t = jnp.dot(ue, wt_ref[...], preferred_element_type=jnp.float32) + bt_ref[...]
    a = jnp.dot(ue, wa_ref[...], preferred_element_type=jnp.float32) + ba_ref[...]
    ier = jnp.concatenate([ie, ie, ie, ie], axis=1)
    q = t * ier
    l = a * ier
    ri = lax.broadcasted_iota(jnp.int32, (KD, KD), 0) // EMB
    ci = lax.broadcasted_iota(jnp.int32, (KD, KD), 1) // EMB
    s = (ri == ci).astype(jnp.float32)
    z = jnp.dot(l, s, preferred_element_type=jnp.float32)
    e = jnp.exp(z)
    denom = jnp.sum(e, axis=1, keepdims=True)
    num = jnp.sum(e * q, axis=1, keepdims=True)
    out_ref[...] = num * float(EMB) / denom + ub + ib


def _tc_mix(ue4, ie4, uo, io, ul, il, ub4, ib4, Wt, bt, Wa, ba):
    grid = (B // BLK,)
    big = pl.BlockSpec((BLK, KD), lambda i: (i, 0))
    col = pl.BlockSpec((BLK, 1), lambda i: (i, 0))
    w = pl.BlockSpec((EMB, KD), lambda i: (0, 0))
    bias = pl.BlockSpec((1, KD), lambda i: (0, 0))
    return pl.pallas_call(
        _tc_mix_body,
        grid=grid,
        in_specs=[big, big, col, col, col, col, big, big, w, bias, w, bias],
        out_specs=col,
        out_shape=jax.ShapeDtypeStruct((B, 1), jnp.float32),
    )(ue4, ie4, uo, io, ul, il, ub4, ib4, Wt, bt, Wa, ba)


@jax.jit
def kernel(user_ids, item_ids, user_emb, item_emb, user_bias, item_bias,
           Wt, bt, Wa, ba):
    uids = user_ids.astype(jnp.int32)
    iids = item_ids.astype(jnp.int32)
    uidq = (uids // PACK).reshape(B // IC, IC)
    iidq = (iids // PACK).reshape(B // IC, IC)
    uemb128 = user_emb.reshape(NROWS * EMB // KD, KD)
    iemb128 = item_emb.reshape(NROWS * EMB // KD, KD)
    ubias128 = jnp.pad(user_bias.reshape(-1), (0, BROWS * KD - NROWS)).reshape(BROWS, KD)
    ibias128 = jnp.pad(item_bias.reshape(-1), (0, BROWS * KD - NROWS)).reshape(BROWS, KD)
    ue4, ie4, ub4, ib4 = _sc_gather(uidq, iidq, uemb128, iemb128,
                                    ubias128, ibias128)
    uo = (uids % PACK).reshape(B, 1)
    io = (iids % PACK).reshape(B, 1)
    ul = (uids % KD).reshape(B, 1)
    il = (iids % KD).reshape(B, 1)
    out = _tc_mix(ue4, ie4, uo, io, ul, il, ub4, ib4,
                  Wt, bt.reshape(1, KD), Wa, ba.reshape(1, KD))
    return out.reshape(-1)
